# baseline (speedup 1.0000x reference)
import jax
import jax.numpy as jnp
from jax import lax
from jax.experimental import pallas as pl
from jax.experimental.pallas import tpu as pltpu
from jax.experimental.pallas import tpu_sc as plsc

X_SIZE = 100000
Y_SIZE = 100000
EMB = 128
NF = 64
NRAW = 512
H0, H1, NATTR = 1024, 512, 256
N_SAMPLES = 128
B = 16384

NC, NSC = 2, 16
NW = NC * NSC
RPW = B // NW
CHUNK = 128
NCHUNK = RPW // CHUNK

MBLK = 2048
CBLK = 4096


def _sc_gather_body(xt, yt, wns, wans, sx1, sy1, tg1, neg2,
                    xv, yv, wnst, wanst, wnsn, wansn,
                    idx_sx, idx_sy, idx_tg, buf, nidx, sem):
    wid = lax.axis_index("s") * NC + lax.axis_index("c")
    base = wid * RPW

    pltpu.sync_copy(sx1.at[pl.ds(base, RPW)], idx_sx)
    pltpu.sync_copy(sy1.at[pl.ds(base, RPW)], idx_sy)
    pltpu.sync_copy(tg1.at[pl.ds(base, RPW)], idx_tg)

    def gather_rows(tab, idx_v, out_hbm):
        for c in range(NCHUNK):
            pltpu.async_copy(tab.at[idx_v.at[pl.ds(c * CHUNK, CHUNK)]],
                             buf.at[pl.ds(c * CHUNK, CHUNK)], sem).wait()
        pltpu.sync_copy(buf, out_hbm.at[pl.ds(base, RPW)])

    gather_rows(xt, idx_sx, xv)
    gather_rows(yt, idx_sy, yv)
    gather_rows(wns, idx_tg, wnst)
    gather_rows(wans, idx_tg, wanst)

    @pl.when(wid == 0)
    def _():
        pltpu.sync_copy(neg2.at[0], nidx)
        pltpu.async_copy(wns.at[nidx], buf.at[pl.ds(0, CHUNK)], sem).wait()
        pltpu.sync_copy(buf.at[pl.ds(0, CHUNK)], wnsn)

    @pl.when(wid == 1)
    def _():
        pltpu.sync_copy(neg2.at[0], nidx)
        pltpu.async_copy(wans.at[nidx], buf.at[pl.ds(0, CHUNK)], sem).wait()
        pltpu.sync_copy(buf.at[pl.ds(0, CHUNK)], wansn)


def _sc_gather(x_table, y_table, W_ns, W_ans, sx3, sy3, tg3, neg2):
    mesh = plsc.VectorSubcoreMesh(core_axis_name="c", subcore_axis_name="s",
                                  num_cores=NC, num_subcores=NSC)
    f32 = jnp.float32
    out_type = (
        jax.ShapeDtypeStruct((B, EMB), f32),
        jax.ShapeDtypeStruct((B, EMB), f32),
        jax.ShapeDtypeStruct((B, EMB), f32),
        jax.ShapeDtypeStruct((B, EMB), f32),
        jax.ShapeDtypeStruct((N_SAMPLES, EMB), f32),
        jax.ShapeDtypeStruct((N_SAMPLES, EMB), f32),
    )
    scratch = [
        pltpu.VMEM((RPW,), jnp.int32),
        pltpu.VMEM((RPW,), jnp.int32),
        pltpu.VMEM((RPW,), jnp.int32),
        pltpu.VMEM((RPW, EMB), f32),
        pltpu.VMEM((CHUNK,), jnp.int32),
        pltpu.SemaphoreType.DMA,
    ]
    return pl.kernel(_sc_gather_body, out_type=out_type, mesh=mesh,
                     scratch_types=scratch)(
        x_table, y_table, W_ns, W_ans, sx3, sy3, tg3, neg2)


def _leaky(x):
    return jnp.maximum(x, 0.2 * x)


def _mlp_body(sa, ta, w0, b0, w1, b1, w2, b2, yP, yQ, Pans, Qans, ya, pa):
    def enc(a_ref):
        h = jnp.dot(a_ref[...], w0[...], preferred_element_type=jnp.float32) + b0[...]
        h = _leaky(h)
        h = jnp.dot(h, w1[...], preferred_element_type=jnp.float32) + b1[...]
        h = _leaky(h)
        h = jnp.dot(h, w2[...], preferred_element_type=jnp.float32) + b2[...]
        return _leaky(h)
    sd = enc(sa)
    td = enc(ta)
    ya[...] = jnp.dot(jnp.dot(sd, yP[...], preferred_element_type=jnp.float32),
                      yQ[...], preferred_element_type=jnp.float32
                      ).astype(jnp.bfloat16)
    pa[...] = jnp.dot(jnp.dot(td, Pans[...], preferred_element_type=jnp.float32),
                      Qans[...], preferred_element_type=jnp.float32
                      ).astype(jnp.bfloat16)


def _mlp(source_attr, target_attr, w0, b0, w1, b1, w2, b2, yP, yQ, Pans, Qans):
    nblk = B // MBLK
    bf16 = jnp.bfloat16
    return pl.pallas_call(
        _mlp_body,
        grid=(nblk,),
        in_specs=[
            pl.BlockSpec((MBLK, NRAW), lambda i: (i, 0)),
            pl.BlockSpec((MBLK, NRAW), lambda i: (i, 0)),
            pl.BlockSpec((NRAW, H0), lambda i: (0, 0)),
            pl.BlockSpec((1, H0), lambda i: (0, 0)),
            pl.BlockSpec((H0, H1), lambda i: (0, 0)),
            pl.BlockSpec((1, H1), lambda i: (0, 0)),
            pl.BlockSpec((H1, NATTR), lambda i: (0, 0)),
            pl.BlockSpec((1, NATTR), lambda i: (0, 0)),
            pl.BlockSpec((NATTR, NF), lambda i: (0, 0)),
            pl.BlockSpec((NF, EMB), lambda i: (0, 0)),
            pl.BlockSpec((NATTR, NF), lambda i: (0, 0)),
            pl.BlockSpec((NF, EMB), lambda i: (0, 0)),
        ],
        out_specs=[
            pl.BlockSpec((MBLK, EMB), lambda i: (i, 0)),
            pl.BlockSpec((MBLK, EMB), lambda i: (i, 0)),
        ],
        out_shape=[
            jax.ShapeDtypeStruct((B, EMB), bf16),
            jax.ShapeDtypeStruct((B, EMB), bf16),
        ],
    )(source_attr, target_attr, w0, b0, w1, b1, w2, b2, yP, yQ, Pans, Qans)


def _log_sigmoid(z):
    return jnp.minimum(z, 0.0) - jnp.log(1.0 + jnp.exp(-jnp.abs(z)))


def _combine_body(ya, pa, xv, yvr, wnst, wanst, wnsn, wansn, out):
    i = pl.program_id(0)

    y_vec = yvr[...] + ya[...].astype(jnp.float32)
    pos_w = wanst[...] + pa[...].astype(jnp.float32)

    xvb = xv[...]
    p1 = jnp.sum(xvb * wnst[...], axis=1, keepdims=True)
    p2 = jnp.sum(y_vec * pos_w, axis=1, keepdims=True)
    pos_logits = p1 + p2

    n1 = lax.dot_general(xvb, wnsn[...], (((1,), (1,)), ((), ())),
                         preferred_element_type=jnp.float32)
    n2 = lax.dot_general(y_vec, wansn[...], (((1,), (1,)), ((), ())),
                         preferred_element_type=jnp.float32)
    neg_logits = n1 + n2

    pos_partial = jnp.sum(_log_sigmoid(pos_logits))
    neg_partial = jnp.sum(_log_sigmoid(-neg_logits))

    rows = lax.broadcasted_iota(jnp.int32, (8, 128), 0)
    cols = lax.broadcasted_iota(jnp.int32, (8, 128), 1)
    val = (jnp.where((rows == 0) & (cols == 0), pos_partial, 0.0)
           + jnp.where((rows == 0) & (cols == 1), neg_partial, 0.0))

    @pl.when(i == 0)
    def _():
        out[...] = jnp.zeros((8, 128), jnp.float32)
    out[...] += val


def _combine(ya, pa, xv, yvr, wnst, wanst, wnsn, wansn):
    nblk = B // CBLK
    return pl.pallas_call(
        _combine_body,
        grid=(nblk,),
        in_specs=[
            pl.BlockSpec((CBLK, EMB), lambda i: (i, 0)),
            pl.BlockSpec((CBLK, EMB), lambda i: (i, 0)),
            pl.BlockSpec((CBLK, EMB), lambda i: (i, 0)),
            pl.BlockSpec((CBLK, EMB), lambda i: (i, 0)),
            pl.BlockSpec((CBLK, EMB), lambda i: (i, 0)),
            pl.BlockSpec((CBLK, EMB), lambda i: (i, 0)),
            pl.BlockSpec((N_SAMPLES, EMB), lambda i: (0, 0)),
            pl.BlockSpec((N_SAMPLES, EMB), lambda i: (0, 0)),
        ],
        out_specs=pl.BlockSpec((8, 128), lambda i: (0, 0)),
        out_shape=jax.ShapeDtypeStruct((8, 128), jnp.float32),
    )(ya, pa, xv, yvr, wnst, wanst, wnsn, wansn)


def kernel(source_x, source_y, source_attr, target, target_attr,
           enc_W0, enc_b0, enc_W1, enc_b1, enc_W2, enc_b2,
           x_table, y_table, y_P, y_Q,
           W_ns, b_ns, W_ans, b_ans, P_ans, Q_ans):
    bf16 = jnp.bfloat16

    neg = jax.random.randint(jax.random.key(1234), (N_SAMPLES,), 0, Y_SIZE)

    i32 = jnp.int32
    sx3 = source_x.astype(i32)
    sy3 = source_y.astype(i32)
    tg3 = target.astype(i32)
    neg2 = neg.astype(i32).reshape(1, N_SAMPLES)

    xv, yvr, wnst, wanst, wnsn, wansn = _sc_gather(
        x_table, y_table, W_ns, W_ans, sx3, sy3, tg3, neg2)

    ya, pa = _mlp(source_attr, target_attr,
                  enc_W0, enc_b0.reshape(1, H0),
                  enc_W1, enc_b1.reshape(1, H1),
                  enc_W2, enc_b2.reshape(1, NATTR),
                  y_P, y_Q, P_ans, Q_ans)

    acc = _combine(ya, pa, xv, yvr, wnst, wanst, wnsn, wansn)

    pos_sum = acc[0, 0]
    neg_sum = acc[0, 1]
    return -(pos_sum / B) - (neg_sum / (B * N_SAMPLES))


# speedup vs baseline: 1.0408x; 1.0408x over previous
import jax
import jax.numpy as jnp
from jax import lax
from jax.experimental import pallas as pl
from jax.experimental.pallas import tpu as pltpu
from jax.experimental.pallas import tpu_sc as plsc

X_SIZE = 100000
Y_SIZE = 100000
EMB = 128
NF = 64
NRAW = 512
H0, H1, NATTR = 1024, 512, 256
N_SAMPLES = 128
B = 16384

NC, NSC = 2, 16
NW = NC * NSC
RPW = B // NW
CHUNK = 128
NCHUNK = RPW // CHUNK

MBLK = 2048
CBLK = 4096


def _sc_gather_body(xt, yt, wns, wans, sx1, sy1, tg1, neg2,
                    xv, yv, wnst, wanst, wnsn, wansn,
                    idx_sx, idx_sy, idx_tg, buf, nidx, sem):
    wid = lax.axis_index("s") * NC + lax.axis_index("c")
    base = wid * RPW

    pltpu.sync_copy(sx1.at[pl.ds(base, RPW)], idx_sx)
    pltpu.sync_copy(sy1.at[pl.ds(base, RPW)], idx_sy)
    pltpu.sync_copy(tg1.at[pl.ds(base, RPW)], idx_tg)

    def gather_rows(tab, idx_v, out_hbm):
        for c in range(NCHUNK):
            pltpu.async_copy(tab.at[idx_v.at[pl.ds(c * CHUNK, CHUNK)]],
                             buf.at[pl.ds(c * CHUNK, CHUNK)], sem).wait()
        pltpu.sync_copy(buf, out_hbm.at[pl.ds(base, RPW)])

    gather_rows(xt, idx_sx, xv)
    gather_rows(yt, idx_sy, yv)
    gather_rows(wns, idx_tg, wnst)
    gather_rows(wans, idx_tg, wanst)

    @pl.when(wid == 0)
    def _():
        pltpu.sync_copy(neg2.at[0], nidx)
        pltpu.async_copy(wns.at[nidx], buf.at[pl.ds(0, CHUNK)], sem).wait()
        pltpu.sync_copy(buf.at[pl.ds(0, CHUNK)], wnsn)

    @pl.when(wid == 1)
    def _():
        pltpu.sync_copy(neg2.at[0], nidx)
        pltpu.async_copy(wans.at[nidx], buf.at[pl.ds(0, CHUNK)], sem).wait()
        pltpu.sync_copy(buf.at[pl.ds(0, CHUNK)], wansn)


def _sc_gather(x_table, y_table, W_ns, W_ans, sx3, sy3, tg3, neg2):
    mesh = plsc.VectorSubcoreMesh(core_axis_name="c", subcore_axis_name="s",
                                  num_cores=NC, num_subcores=NSC)
    f32 = jnp.float32
    out_type = (
        jax.ShapeDtypeStruct((B, EMB), f32),
        jax.ShapeDtypeStruct((B, EMB), f32),
        jax.ShapeDtypeStruct((B, EMB), f32),
        jax.ShapeDtypeStruct((B, EMB), f32),
        jax.ShapeDtypeStruct((N_SAMPLES, EMB), f32),
        jax.ShapeDtypeStruct((N_SAMPLES, EMB), f32),
    )
    scratch = [
        pltpu.VMEM((RPW,), jnp.int32),
        pltpu.VMEM((RPW,), jnp.int32),
        pltpu.VMEM((RPW,), jnp.int32),
        pltpu.VMEM((RPW, EMB), f32),
        pltpu.VMEM((CHUNK,), jnp.int32),
        pltpu.SemaphoreType.DMA,
    ]
    return pl.kernel(_sc_gather_body, out_type=out_type, mesh=mesh,
                     scratch_types=scratch)(
        x_table, y_table, W_ns, W_ans, sx3, sy3, tg3, neg2)


def _leaky(x):
    return jnp.maximum(x, 0.2 * x)


def _mlp_body(sa, ta, w0, b0, w1, b1, w2, b2, yPQ, PQans, ya, pa):
    def enc(a_ref):
        h = jnp.dot(a_ref[...], w0[...], preferred_element_type=jnp.float32) + b0[...]
        h = _leaky(h)
        h = jnp.dot(h, w1[...], preferred_element_type=jnp.float32) + b1[...]
        h = _leaky(h)
        h = jnp.dot(h, w2[...], preferred_element_type=jnp.float32) + b2[...]
        return _leaky(h)
    sd = enc(sa)
    td = enc(ta)
    ya[...] = jnp.dot(sd, yPQ[...],
                      preferred_element_type=jnp.float32).astype(jnp.bfloat16)
    pa[...] = jnp.dot(td, PQans[...],
                      preferred_element_type=jnp.float32).astype(jnp.bfloat16)


def _mlp(source_attr, target_attr, w0, b0, w1, b1, w2, b2, yPQ, PQans):
    nblk = B // MBLK
    bf16 = jnp.bfloat16
    return pl.pallas_call(
        _mlp_body,
        grid=(nblk,),
        in_specs=[
            pl.BlockSpec((MBLK, NRAW), lambda i: (i, 0)),
            pl.BlockSpec((MBLK, NRAW), lambda i: (i, 0)),
            pl.BlockSpec((NRAW, H0), lambda i: (0, 0)),
            pl.BlockSpec((1, H0), lambda i: (0, 0)),
            pl.BlockSpec((H0, H1), lambda i: (0, 0)),
            pl.BlockSpec((1, H1), lambda i: (0, 0)),
            pl.BlockSpec((H1, NATTR), lambda i: (0, 0)),
            pl.BlockSpec((1, NATTR), lambda i: (0, 0)),
            pl.BlockSpec((NATTR, EMB), lambda i: (0, 0)),
            pl.BlockSpec((NATTR, EMB), lambda i: (0, 0)),
        ],
        out_specs=[
            pl.BlockSpec((MBLK, EMB), lambda i: (i, 0)),
            pl.BlockSpec((MBLK, EMB), lambda i: (i, 0)),
        ],
        out_shape=[
            jax.ShapeDtypeStruct((B, EMB), bf16),
            jax.ShapeDtypeStruct((B, EMB), bf16),
        ],
    )(source_attr, target_attr, w0, b0, w1, b1, w2, b2, yPQ, PQans)


def _log_sigmoid(z):
    return jnp.minimum(z, 0.0) - jnp.log(1.0 + jnp.exp(-jnp.abs(z)))


def _combine_body(ya, pa, xv, yvr, wnst, wanst, wnsn, wansn, out):
    i = pl.program_id(0)

    y_vec = yvr[...] + ya[...].astype(jnp.float32)
    pos_w = wanst[...] + pa[...].astype(jnp.float32)

    xvb = xv[...]
    p1 = jnp.sum(xvb * wnst[...], axis=1, keepdims=True)
    p2 = jnp.sum(y_vec * pos_w, axis=1, keepdims=True)
    pos_logits = p1 + p2

    n1 = lax.dot_general(xvb, wnsn[...], (((1,), (1,)), ((), ())),
                         preferred_element_type=jnp.float32)
    n2 = lax.dot_general(y_vec, wansn[...], (((1,), (1,)), ((), ())),
                         preferred_element_type=jnp.float32)
    neg_logits = n1 + n2

    pos_partial = jnp.sum(_log_sigmoid(pos_logits))
    neg_partial = jnp.sum(_log_sigmoid(-neg_logits))

    rows = lax.broadcasted_iota(jnp.int32, (8, 128), 0)
    cols = lax.broadcasted_iota(jnp.int32, (8, 128), 1)
    val = (jnp.where((rows == 0) & (cols == 0), pos_partial, 0.0)
           + jnp.where((rows == 0) & (cols == 1), neg_partial, 0.0))

    @pl.when(i == 0)
    def _():
        out[...] = jnp.zeros((8, 128), jnp.float32)
    out[...] += val


def _combine(ya, pa, xv, yvr, wnst, wanst, wnsn, wansn):
    nblk = B // CBLK
    return pl.pallas_call(
        _combine_body,
        grid=(nblk,),
        in_specs=[
            pl.BlockSpec((CBLK, EMB), lambda i: (i, 0)),
            pl.BlockSpec((CBLK, EMB), lambda i: (i, 0)),
            pl.BlockSpec((CBLK, EMB), lambda i: (i, 0)),
            pl.BlockSpec((CBLK, EMB), lambda i: (i, 0)),
            pl.BlockSpec((CBLK, EMB), lambda i: (i, 0)),
            pl.BlockSpec((CBLK, EMB), lambda i: (i, 0)),
            pl.BlockSpec((N_SAMPLES, EMB), lambda i: (0, 0)),
            pl.BlockSpec((N_SAMPLES, EMB), lambda i: (0, 0)),
        ],
        out_specs=pl.BlockSpec((8, 128), lambda i: (0, 0)),
        out_shape=jax.ShapeDtypeStruct((8, 128), jnp.float32),
    )(ya, pa, xv, yvr, wnst, wanst, wnsn, wansn)


def kernel(source_x, source_y, source_attr, target, target_attr,
           enc_W0, enc_b0, enc_W1, enc_b1, enc_W2, enc_b2,
           x_table, y_table, y_P, y_Q,
           W_ns, b_ns, W_ans, b_ans, P_ans, Q_ans):
    bf16 = jnp.bfloat16

    neg = jax.random.randint(jax.random.key(1234), (N_SAMPLES,), 0, Y_SIZE)

    i32 = jnp.int32
    sx3 = source_x.astype(i32)
    sy3 = source_y.astype(i32)
    tg3 = target.astype(i32)
    neg2 = neg.astype(i32).reshape(1, N_SAMPLES)

    xv, yvr, wnst, wanst, wnsn, wansn = _sc_gather(
        x_table, y_table, W_ns, W_ans, sx3, sy3, tg3, neg2)

    yPQ = jnp.dot(y_P, y_Q, preferred_element_type=jnp.float32)
    PQans = jnp.dot(P_ans, Q_ans, preferred_element_type=jnp.float32)

    ya, pa = _mlp(source_attr, target_attr,
                  enc_W0, enc_b0.reshape(1, H0),
                  enc_W1, enc_b1.reshape(1, H1),
                  enc_W2, enc_b2.reshape(1, NATTR),
                  yPQ, PQans)

    acc = _combine(ya, pa, xv, yvr, wnst, wanst, wnsn, wansn)

    pos_sum = acc[0, 0]
    neg_sum = acc[0, 1]
    return -(pos_sum / B) - (neg_sum / (B * N_SAMPLES))
